# Initial kernel scaffold; baseline (speedup 1.0000x reference)
import jax
import jax.numpy as jnp
from jax.experimental import pallas as pl


def kernel(input_batch, weight):
    raise NotImplementedError("write your pallas kernel here")


# speedup vs baseline: 1.4486x; 1.4486x over previous
import functools

import jax
import jax.numpy as jnp
from jax import lax
from jax.experimental import pallas as pl
from jax.experimental.pallas import tpu as pltpu
from jax.experimental.pallas import tpu_sc as plsc

NUM_EMBEDDINGS = 1000000
EMBED_DIM = 32
BATCH = 4096
SEQ = 200
TOTAL = BATCH * SEQ

NC = 2
NS = 16
NW = NC * NS
LANES = 16

B_PER_W = TOTAL // NW
CHUNK = 1024
N_CHUNKS = B_PER_W // CHUNK
GATHER_W = 128
G_PER_CHUNK = CHUNK // GATHER_W


def _body(table_hbm, idx_hbm, out_hbm, idx_v, rows_v, sem):
  wid = lax.axis_index("s") * NC + lax.axis_index("c")
  idx_row0 = wid * (B_PER_W // GATHER_W)
  out_row0 = wid * B_PER_W

  @pl.loop(0, N_CHUNKS)
  def _chunk(c):
    pltpu.sync_copy(idx_hbm.at[pl.ds(idx_row0 + c * G_PER_CHUNK, G_PER_CHUNK)],
                    idx_v)

    copies = []
    for j in range(G_PER_CHUNK):
      copies.append(
          pltpu.async_copy(table_hbm.at[idx_v.at[j]],
                           rows_v.at[pl.ds(j * GATHER_W, GATHER_W)], sem))

    m = jnp.full((LANES,), NUM_EMBEDDINGS, dtype=jnp.int32)
    for j in range(G_PER_CHUNK):
      for k in range(GATHER_W // LANES):
        m = jnp.minimum(m, idx_v.at[j][pl.ds(k * LANES, LANES)])
    pad_cnt = plsc.all_reduce_population_count(m == 0)
    has_pad = pad_cnt[0] > 0

    for cp in copies:
      cp.wait()

    @pl.when(has_pad)
    def _fixup():
      zeros16 = jnp.zeros((LANES,), dtype=jnp.float32)
      lane = lax.iota(jnp.int32, LANES)

      @pl.loop(0, G_PER_CHUNK)
      def _j(j):
        @pl.loop(0, GATHER_W // LANES)
        def _k(k):
          v = idx_v.at[j][pl.ds(k * LANES, LANES)]
          msk = v == 0
          row_ids = lane + (j * GATHER_W + k * LANES)

          @pl.loop(0, EMBED_DIM)
          def _col(col):
            col_ids = jnp.full((LANES,), 0, dtype=jnp.int32) + col
            plsc.store_scatter(rows_v, [row_ids, col_ids], zeros16, mask=msk)

    pltpu.sync_copy(rows_v, out_hbm.at[pl.ds(out_row0 + c * CHUNK, CHUNK)])


@jax.jit
def _lookup(idx2d, weight):
  mesh = plsc.VectorSubcoreMesh(core_axis_name="c", subcore_axis_name="s")
  f = pl.kernel(
      _body,
      out_type=jax.ShapeDtypeStruct((TOTAL, EMBED_DIM), jnp.float32),
      mesh=mesh,
      scratch_types=[
          pltpu.VMEM((G_PER_CHUNK, GATHER_W), jnp.int32),
          pltpu.VMEM((CHUNK, EMBED_DIM), jnp.float32),
          pltpu.SemaphoreType.DMA,
      ],
      compiler_params=pltpu.CompilerParams(
          needs_layout_passes=False, use_tc_tiling_on_sc=False),
  )
  return f(weight, idx2d)


def kernel(input_batch, weight):
  idx2d = input_batch.reshape(TOTAL // GATHER_W, GATHER_W)
  out = _lookup(idx2d, weight)
  return out.reshape(BATCH, SEQ, EMBED_DIM)


# speedup vs baseline: 1.4988x; 1.0347x over previous
import jax
import jax.numpy as jnp
from jax import lax
from jax.experimental import pallas as pl
from jax.experimental.pallas import tpu as pltpu
from jax.experimental.pallas import tpu_sc as plsc

NUM_EMBEDDINGS = 1000000
EMBED_DIM = 32
BATCH = 4096
SEQ = 200
TOTAL = BATCH * SEQ

NC = 2
NS = 16
NW = NC * NS
LANES = 16

B_PER_W = TOTAL // NW
GATHER_W = 128
IDX_ROWS = B_PER_W // GATHER_W
CHUNK = 640
G_PER_CHUNK = CHUNK // GATHER_W
N_CHUNKS = B_PER_W // CHUNK
NBUF = 4


def _body(table_hbm, idx_hbm, out_hbm, idx_v, rows_v, gsem, osem):
  wid = lax.axis_index("s") * NC + lax.axis_index("c")
  out_row0 = wid * B_PER_W

  pltpu.sync_copy(idx_hbm.at[pl.ds(wid * IDX_ROWS, IDX_ROWS)], idx_v)

  def fire_gathers(c, b):
    for j in range(G_PER_CHUNK):
      pltpu.async_copy(table_hbm.at[idx_v.at[c * G_PER_CHUNK + j]],
                       rows_v.at[b].at[pl.ds(j * GATHER_W, GATHER_W)],
                       gsem.at[b])

  def drain(sem_ref, vmem_ref):
    pltpu.make_async_copy(out_hbm.at[pl.ds(0, CHUNK)], vmem_ref, sem_ref).wait()

  for b in range(NBUF - 1):
    fire_gathers(b, b)

  @pl.loop(0, N_CHUNKS, step=NBUF)
  def _ring(c0):
    for b in range(NBUF):
      c = c0 + b
      b3 = (b + NBUF - 1) % NBUF

      @pl.when(c > 0)
      def _drain_prev_out():
        drain(osem.at[b3], rows_v.at[b3])

      @pl.when(c + NBUF - 1 < N_CHUNKS)
      def _fire_next():
        fire_gathers(c + NBUF - 1, b3)

      m = jnp.full((LANES,), NUM_EMBEDDINGS, dtype=jnp.int32)
      for j in range(G_PER_CHUNK):
        for k in range(GATHER_W // LANES):
          m = jnp.minimum(m, idx_v.at[c * G_PER_CHUNK + j][pl.ds(k * LANES,
                                                                 LANES)])
      has_pad = plsc.all_reduce_population_count(m == 0)[0] > 0

      drain(gsem.at[b], rows_v.at[b])

      @pl.when(has_pad)
      def _fixup():
        zeros16 = jnp.zeros((LANES,), dtype=jnp.float32)
        lane = lax.iota(jnp.int32, LANES)

        @pl.loop(0, G_PER_CHUNK)
        def _j(j):
          @pl.loop(0, GATHER_W // LANES)
          def _k(k):
            v = idx_v.at[c * G_PER_CHUNK + j][pl.ds(k * LANES, LANES)]
            msk = v == 0
            row_ids = lane + (j * GATHER_W + k * LANES)

            @pl.loop(0, EMBED_DIM)
            def _col(col):
              col_ids = jnp.full((LANES,), 0, dtype=jnp.int32) + col
              plsc.store_scatter(rows_v.at[b], [row_ids, col_ids], zeros16,
                                 mask=msk)

      pltpu.async_copy(rows_v.at[b],
                       out_hbm.at[pl.ds(out_row0 + c * CHUNK, CHUNK)],
                       osem.at[b])

  drain(osem.at[(N_CHUNKS - 1) % NBUF], rows_v.at[(N_CHUNKS - 1) % NBUF])


@jax.jit
def _lookup(idx2d, weight):
  mesh = plsc.VectorSubcoreMesh(core_axis_name="c", subcore_axis_name="s")
  f = pl.kernel(
      _body,
      out_type=jax.ShapeDtypeStruct((TOTAL, EMBED_DIM), jnp.float32),
      mesh=mesh,
      scratch_types=[
          pltpu.VMEM((IDX_ROWS, GATHER_W), jnp.int32),
          pltpu.VMEM((NBUF, CHUNK, EMBED_DIM), jnp.float32),
          pltpu.SemaphoreType.DMA((NBUF,)),
          pltpu.SemaphoreType.DMA((NBUF,)),
      ],
      compiler_params=pltpu.CompilerParams(
          needs_layout_passes=False, use_tc_tiling_on_sc=False),
  )
  return f(weight, idx2d)


def kernel(input_batch, weight):
  idx2d = input_batch.reshape(TOTAL // GATHER_W, GATHER_W)
  out = _lookup(idx2d, weight)
  return out.reshape(BATCH, SEQ, EMBED_DIM)
